# Initial kernel scaffold; baseline (speedup 1.0000x reference)
import jax
import jax.numpy as jnp
from jax.experimental import pallas as pl


def kernel(x, edge_index, batch, Wl0, Wr0, att0, b0, g0, be0, Wl1, Wr1, att1, b1, g1, be1, Wl2, Wr2, att2, b2, g2, be2, fc1_w, fc1_b, gfc, bfc, fc2_w, fc2_b, gemb, bemb):
    raise NotImplementedError("write your pallas kernel here")


# speedup vs baseline: 23.5412x; 23.5412x over previous
import functools

import jax
import jax.numpy as jnp
from jax import lax
from jax.experimental import pallas as pl
from jax.experimental.pallas import tpu as pltpu
from jax.experimental.pallas import tpu_sc as plsc

N = 10000
E = 160000
H = 4
C = 64
HC = 256
HALF = 128
G = 64
HID = 64
EMB = 256

NS = 16
K = 16
EPT = E // NS
NCH = EPT // K
RPT = N // NS
ACC_W = 144
ACC_R = 10240
ZPT = ACC_R // NS
GROW = N


def _edge_body(xl2, xr2, srce, dste, attf, zrows, out,
               src_v, dst_v, xlv, xrv, stage, attv, acc, sem1, sem2):
    cid = lax.axis_index("c")
    sid = lax.axis_index("s")
    pltpu.sync_copy(zrows, acc.at[pl.ds(sid * ZPT, ZPT)])
    base = sid * EPT
    pltpu.sync_copy(srce.at[pl.ds(base, EPT)], src_v)
    pltpu.sync_copy(dste.at[pl.ds(base, EPT)], dst_v)
    pltpu.sync_copy(attf.at[pl.ds(cid * HALF, HALF)], attv)
    plsc.subcore_barrier()

    offn = cid * N
    att_r = [attv[pl.ds(16 * c, 16)] for c in range(8)]
    lane = lax.iota(jnp.int32, 16)

    def chunk(i, carry):
        off = i * K
        sv = src_v[pl.ds(off, K)]
        dv = dst_v[pl.ds(off, K)]
        c1 = pltpu.async_copy(xl2.at[sv + offn], xlv, sem1)
        c2 = pltpu.async_copy(xr2.at[dv + offn], xrv, sem2)
        c1.wait()
        c2.wait()
        for e in range(K):
            xle = [xlv[e, pl.ds(16 * c, 16)] for c in range(8)]
            ts = []
            for c in range(8):
                z = xle[c] + xrv[e, pl.ds(16 * c, 16)]
                z = jnp.maximum(z, 0.2 * z)
                ts.append(z * att_r[c])
            s0 = jnp.sum((ts[0] + ts[1]) + (ts[2] + ts[3]))
            s1 = jnp.sum((ts[4] + ts[5]) + (ts[6] + ts[7]))
            ae0 = jnp.exp(lax.broadcast(s0, (16,)))
            ae1 = jnp.exp(lax.broadcast(s1, (16,)))
            for c in range(4):
                stage[e, pl.ds(16 * c, 16)] = xle[c] * ae0
            for c in range(4, 8):
                stage[e, pl.ds(16 * c, 16)] = xle[c] * ae1
            aev = jnp.where(lane == 0, ae0, jnp.where(lane == 1, ae1, 0.0))
            stage[e, pl.ds(HALF, 16)] = aev
        rank = jnp.zeros((16,), jnp.int32)
        for k in range(1, K):
            rot = dv.at[jnp.maximum(lane - k, 0)].get(mode="promise_in_bounds")
            rank = rank + jnp.where((lane >= k) & (rot == dv), 1, 0)
        maxr = jnp.max(rank)

        def scat(p, c):
            idxp = jnp.where(rank == p, dv, GROW)
            pltpu.sync_copy(stage, acc.at[idxp], add=True)
            return c

        lax.fori_loop(0, maxr + 1, scat, 0)
        return carry

    lax.fori_loop(0, NCH, chunk, 0)
    plsc.subcore_barrier()
    pltpu.sync_copy(acc.at[pl.ds(sid * RPT, RPT)],
                    out.at[cid, pl.ds(sid * RPT, RPT)])


_edge_pass = pl.kernel(
    _edge_body,
    out_type=jax.ShapeDtypeStruct((2, N, ACC_W), jnp.float32),
    mesh=plsc.VectorSubcoreMesh(core_axis_name="c", subcore_axis_name="s"),
    scratch_types=[
        pltpu.VMEM((EPT,), jnp.int32),
        pltpu.VMEM((EPT,), jnp.int32),
        pltpu.VMEM((K, HALF), jnp.float32),
        pltpu.VMEM((K, HALF), jnp.float32),
        pltpu.VMEM((K, ACC_W), jnp.float32),
        pltpu.VMEM((HALF,), jnp.float32),
        pltpu.VMEM_SHARED((ACC_R, ACC_W), jnp.float32),
        pltpu.SemaphoreType.DMA,
        pltpu.SemaphoreType.DMA,
    ],
    compiler_params=pltpu.CompilerParams(use_tc_tiling_on_sc=False,
                                         needs_layout_passes=False),
)


_PREB = 1000


def _pre_body(h_ref, wl_ref, wr_ref, xl2_ref, xr2_ref):
    h = h_ref[...]
    xl = jnp.dot(h, wl_ref[...], preferred_element_type=jnp.float32)
    xr = jnp.dot(h, wr_ref[...], preferred_element_type=jnp.float32)
    xl2_ref[0] = xl[:, :HALF]
    xl2_ref[1] = xl[:, HALF:]
    xr2_ref[0] = xr[:, :HALF]
    xr2_ref[1] = xr[:, HALF:]


def _pre(h, wl, wr):
    din = h.shape[1]
    return pl.pallas_call(
        _pre_body,
        grid=(N // _PREB,),
        in_specs=[pl.BlockSpec((_PREB, din), lambda i: (i, 0)),
                  pl.BlockSpec((din, HC), lambda i: (0, 0)),
                  pl.BlockSpec((din, HC), lambda i: (0, 0))],
        out_specs=[pl.BlockSpec((2, _PREB, HALF), lambda i: (0, i, 0)),
                   pl.BlockSpec((2, _PREB, HALF), lambda i: (0, i, 0))],
        out_shape=[jax.ShapeDtypeStruct((2, N, HALF), jnp.float32),
                   jax.ShapeDtypeStruct((2, N, HALF), jnp.float32)],
    )(h, wl, wr)


def _finish_layer(sc):
    num0 = sc[0][:, :HALF]
    num1 = sc[1][:, :HALF]
    d00 = sc[0][:, HALF:HALF + 1] + 1e-16
    d01 = sc[0][:, HALF + 1:HALF + 2] + 1e-16
    d10 = sc[1][:, HALF:HALF + 1] + 1e-16
    d11 = sc[1][:, HALF + 1:HALF + 2] + 1e-16
    den0 = jnp.concatenate([jnp.broadcast_to(d00, (N, C)), jnp.broadcast_to(d01, (N, C))], 1)
    den1 = jnp.concatenate([jnp.broadcast_to(d10, (N, C)), jnp.broadcast_to(d11, (N, C))], 1)
    return jnp.concatenate([num0 / den0, num1 / den1], 1)


def _bn(x, g, b):
    mu = jnp.mean(x, 0, keepdims=True)
    var = jnp.mean((x - mu) * (x - mu), 0, keepdims=True)
    return g * (x - mu) / jnp.sqrt(var + 1e-5) + b


def _epi_body(sc_ref, b_ref, g_ref, be_ref, h_ref):
    h = _finish_layer(sc_ref) + b_ref[...]
    hn = _bn(h, g_ref[...], be_ref[...])
    h_ref[...] = jnp.where(hn > 0, hn, 0.01 * hn)


def _epi(sc, b, g, be):
    return pl.pallas_call(
        _epi_body,
        out_shape=jax.ShapeDtypeStruct((N, HC), jnp.float32),
    )(sc, b.reshape(1, HC), g.reshape(1, HC), be.reshape(1, HC))


def _fin_body(sc_ref, b_ref, g_ref, be_ref, batch_ref, fc1w_ref, fc1b_ref,
              gfc_ref, bfc_ref, fc2w_ref, fc2b_ref, gemb_ref, bemb_ref, out_ref):
    h = _finish_layer(sc_ref) + b_ref[...]
    hn = _bn(h, g_ref[...], be_ref[...])
    h = jnp.where(hn > 0, hn, 0.01 * hn)
    gi = lax.broadcasted_iota(jnp.int32, (G, N), 0)
    onehot = (gi == batch_ref[...]).astype(jnp.float32)
    sums = jnp.dot(onehot, h, preferred_element_type=jnp.float32, precision=lax.Precision.HIGHEST)
    cnt = jnp.sum(onehot, 1, keepdims=True)
    hg = sums / jnp.maximum(cnt, 1.0)
    z = jnp.dot(hg, fc1w_ref[...], preferred_element_type=jnp.float32, precision=lax.Precision.HIGHEST) + fc1b_ref[...]
    z = _bn(z, gfc_ref[...], bfc_ref[...])
    z = jnp.where(z > 0, z, 0.01 * z)
    z = jnp.dot(z, fc2w_ref[...], preferred_element_type=jnp.float32, precision=lax.Precision.HIGHEST) + fc2b_ref[...]
    z = _bn(z, gemb_ref[...], bemb_ref[...])
    nrm = jnp.maximum(jnp.sqrt(jnp.sum(z * z, 1, keepdims=True)), 1e-12)
    out_ref[...] = z / nrm


def _pool_body(h_ref, batch_ref, hg_ref):
    h = h_ref[...]
    gi = lax.broadcasted_iota(jnp.int32, (G, N), 0)
    onehot = (gi == batch_ref[...]).astype(jnp.float32)
    sums = jnp.dot(onehot, h, preferred_element_type=jnp.float32, precision=lax.Precision.HIGHEST)
    cnt = jnp.sum(onehot, 1, keepdims=True)
    hg_ref[...] = sums / jnp.maximum(cnt, 1.0)


def _pool(h, batch):
    return pl.pallas_call(
        _pool_body,
        out_shape=jax.ShapeDtypeStruct((G, HC), jnp.float32),
    )(h, batch.reshape(1, N))


def _head_body(hg_ref, fc1w_ref, fc1b_ref, gfc_ref, bfc_ref, fc2w_ref, fc2b_ref,
               gemb_ref, bemb_ref, out_ref):
    z = jnp.dot(hg_ref[...], fc1w_ref[...], preferred_element_type=jnp.float32) + fc1b_ref[...]
    z = _bn(z, gfc_ref[...], bfc_ref[...])
    z = jnp.where(z > 0, z, 0.01 * z)
    z = jnp.dot(z, fc2w_ref[...], preferred_element_type=jnp.float32) + fc2b_ref[...]
    z = _bn(z, gemb_ref[...], bemb_ref[...])
    nrm = jnp.maximum(jnp.sqrt(jnp.sum(z * z, 1, keepdims=True)), 1e-12)
    out_ref[...] = z / nrm


def _head(hg, fc1w, fc1b, gfc, bfc, fc2w, fc2b, gemb, bemb):
    return pl.pallas_call(
        _head_body,
        out_shape=jax.ShapeDtypeStruct((G, EMB), jnp.float32),
    )(hg, fc1w, fc1b.reshape(1, HID), gfc.reshape(1, HID), bfc.reshape(1, HID),
      fc2w, fc2b.reshape(1, EMB), gemb.reshape(1, EMB), bemb.reshape(1, EMB))


def _fin(sc, b, g, be, batch, fc1w, fc1b, gfc, bfc, fc2w, fc2b, gemb, bemb):
    return pl.pallas_call(
        _fin_body,
        out_shape=jax.ShapeDtypeStruct((G, EMB), jnp.float32),
    )(sc, b.reshape(1, HC), g.reshape(1, HC), be.reshape(1, HC),
      batch.reshape(1, N), fc1w, fc1b.reshape(1, HID), gfc.reshape(1, HID),
      bfc.reshape(1, HID), fc2w, fc2b.reshape(1, EMB), gemb.reshape(1, EMB),
      bemb.reshape(1, EMB))


def kernel(x, edge_index, batch, Wl0, Wr0, att0, b0, g0, be0, Wl1, Wr1, att1, b1, g1, be1,
           Wl2, Wr2, att2, b2, g2, be2, fc1_w, fc1_b, gfc, bfc, fc2_w, fc2_b, gemb, bemb):
    src = edge_index[0]
    dst = edge_index[1]
    zrows = jnp.zeros((ZPT, ACC_W), jnp.float32)
    layers = ((Wl0, Wr0, att0, b0, g0, be0),
              (Wl1, Wr1, att1, b1, g1, be1),
              (Wl2, Wr2, att2, b2, g2, be2))
    h = x
    for li, (Wl, Wr, att, bb, g, be) in enumerate(layers):
        xl2, xr2 = _pre(h, Wl, Wr)
        sc = _edge_pass(xl2.reshape(2 * N, HALF), xr2.reshape(2 * N, HALF),
                        src, dst, att.reshape(HC), zrows)
        h = _epi(sc, bb, g, be)
    hg = _pool(h, batch)
    return _head(hg, fc1_w, fc1_b, gfc, bfc, fc2_w, fc2_b, gemb, bemb)


# speedup vs baseline: 42.1054x; 1.7886x over previous
import functools

import jax
import jax.numpy as jnp
from jax import lax
from jax.experimental import pallas as pl
from jax.experimental.pallas import tpu as pltpu
from jax.experimental.pallas import tpu_sc as plsc

N = 10000
E = 160000
H = 4
C = 64
HC = 256
HALF = 128
G = 64
HID = 64
EMB = 256

NS = 16
K = 16
EPT = E // NS
NCH = EPT // K
RPT = N // NS
ACC_W = 144
ACC_R = 10240
ZPT = ACC_R // NS
GROW = N


def _edge_body(xl2, xr2, srce, dste, attf, zrows, out,
               src_v, dst_v, xlv, xrv, stage, xlv1, xrv1, stage1, attv, acc,
               sem1, sem2, sem3, sem4):
    cid = lax.axis_index("c")
    sid = lax.axis_index("s")
    pltpu.sync_copy(zrows, acc.at[pl.ds(sid * ZPT, ZPT)])
    base = sid * EPT
    pltpu.sync_copy(srce.at[pl.ds(base, EPT)], src_v)
    pltpu.sync_copy(dste.at[pl.ds(base, EPT)], dst_v)
    pltpu.sync_copy(attf.at[pl.ds(cid * HALF, HALF)], attv)
    plsc.subcore_barrier()

    offn = cid * N
    att_r = [attv[pl.ds(16 * c, 16)] for c in range(8)]
    lane = lax.iota(jnp.int32, 16)

    bufs = ((xlv, xrv, stage, sem1, sem2), (xlv1, xrv1, stage1, sem3, sem4))

    def gidx(i):
        off = i * K
        sv = src_v[pl.ds(off, K)]
        dv = dst_v[pl.ds(off, K)]
        return sv, dv

    def issue(i, b):
        sv, dv = gidx(i)
        pltpu.async_copy(xl2.at[sv + offn], b[0], b[3])
        pltpu.async_copy(xr2.at[dv + offn], b[1], b[4])

    def waitg(i, b):
        sv, dv = gidx(i)
        pltpu.make_async_copy(xl2.at[sv + offn], b[0], b[3]).wait()
        pltpu.make_async_copy(xr2.at[dv + offn], b[1], b[4]).wait()

    def compute_scatter(i, b):
        bx, br, stg = b[0], b[1], b[2]
        _, dv = gidx(i)
        for e in range(K):
            xle = [bx[e, pl.ds(16 * c, 16)] for c in range(8)]
            ts = []
            for c in range(8):
                z = xle[c] + br[e, pl.ds(16 * c, 16)]
                z = jnp.maximum(z, 0.2 * z)
                ts.append(z * att_r[c])
            s0 = jnp.sum((ts[0] + ts[1]) + (ts[2] + ts[3]))
            s1 = jnp.sum((ts[4] + ts[5]) + (ts[6] + ts[7]))
            ae0 = jnp.exp(lax.broadcast(s0, (16,)))
            ae1 = jnp.exp(lax.broadcast(s1, (16,)))
            for c in range(4):
                stg[e, pl.ds(16 * c, 16)] = xle[c] * ae0
            for c in range(4, 8):
                stg[e, pl.ds(16 * c, 16)] = xle[c] * ae1
            aev = jnp.where(lane == 0, ae0, jnp.where(lane == 1, ae1, 0.0))
            stg[e, pl.ds(HALF, 16)] = aev
        rank = jnp.zeros((16,), jnp.int32)
        for k in range(1, K):
            rot = dv.at[jnp.maximum(lane - k, 0)].get(mode="promise_in_bounds")
            rank = rank + jnp.where((lane >= k) & (rot == dv), 1, 0)
        maxr = jnp.max(rank)

        def scat(p, c):
            idxp = jnp.where(rank == p, dv, GROW)
            pltpu.sync_copy(stg, acc.at[idxp], add=True)
            return c

        lax.fori_loop(0, maxr + 1, scat, 0)

    issue(0, bufs[0])

    def loop(j, carry):
        i0 = 2 * j
        issue(i0 + 1, bufs[1])
        waitg(i0, bufs[0])
        compute_scatter(i0, bufs[0])
        issue(i0 + 2, bufs[0])
        waitg(i0 + 1, bufs[1])
        compute_scatter(i0 + 1, bufs[1])
        return carry

    lax.fori_loop(0, NCH // 2, loop, 0)
    waitg(NCH - 1, bufs[0])
    compute_scatter(NCH - 1, bufs[0])
    plsc.subcore_barrier()
    pltpu.sync_copy(acc.at[pl.ds(sid * RPT, RPT)],
                    out.at[cid, pl.ds(sid * RPT, RPT)])


_edge_pass = pl.kernel(
    _edge_body,
    out_type=jax.ShapeDtypeStruct((2, N, ACC_W), jnp.float32),
    mesh=plsc.VectorSubcoreMesh(core_axis_name="c", subcore_axis_name="s"),
    scratch_types=[
        pltpu.VMEM((EPT,), jnp.int32),
        pltpu.VMEM((EPT,), jnp.int32),
        pltpu.VMEM((K, HALF), jnp.float32),
        pltpu.VMEM((K, HALF), jnp.float32),
        pltpu.VMEM((K, ACC_W), jnp.float32),
        pltpu.VMEM((K, HALF), jnp.float32),
        pltpu.VMEM((K, HALF), jnp.float32),
        pltpu.VMEM((K, ACC_W), jnp.float32),
        pltpu.VMEM((HALF,), jnp.float32),
        pltpu.VMEM_SHARED((ACC_R, ACC_W), jnp.float32),
        pltpu.SemaphoreType.DMA,
        pltpu.SemaphoreType.DMA,
        pltpu.SemaphoreType.DMA,
        pltpu.SemaphoreType.DMA,
    ],
    compiler_params=pltpu.CompilerParams(use_tc_tiling_on_sc=False,
                                         needs_layout_passes=False),
)


_PREB = 1000


def _pre_body(h_ref, wl_ref, wr_ref, xl2_ref, xr2_ref):
    h = h_ref[...]
    xl = jnp.dot(h, wl_ref[...], preferred_element_type=jnp.float32)
    xr = jnp.dot(h, wr_ref[...], preferred_element_type=jnp.float32)
    xl2_ref[0] = xl[:, :HALF]
    xl2_ref[1] = xl[:, HALF:]
    xr2_ref[0] = xr[:, :HALF]
    xr2_ref[1] = xr[:, HALF:]


def _pre(h, wl, wr):
    din = h.shape[1]
    return pl.pallas_call(
        _pre_body,
        grid=(N // _PREB,),
        in_specs=[pl.BlockSpec((_PREB, din), lambda i: (i, 0)),
                  pl.BlockSpec((din, HC), lambda i: (0, 0)),
                  pl.BlockSpec((din, HC), lambda i: (0, 0))],
        out_specs=[pl.BlockSpec((2, _PREB, HALF), lambda i: (0, i, 0)),
                   pl.BlockSpec((2, _PREB, HALF), lambda i: (0, i, 0))],
        out_shape=[jax.ShapeDtypeStruct((2, N, HALF), jnp.float32),
                   jax.ShapeDtypeStruct((2, N, HALF), jnp.float32)],
    )(h, wl, wr)


def _finish_layer(sc):
    num0 = sc[0][:, :HALF]
    num1 = sc[1][:, :HALF]
    d00 = sc[0][:, HALF:HALF + 1] + 1e-16
    d01 = sc[0][:, HALF + 1:HALF + 2] + 1e-16
    d10 = sc[1][:, HALF:HALF + 1] + 1e-16
    d11 = sc[1][:, HALF + 1:HALF + 2] + 1e-16
    den0 = jnp.concatenate([jnp.broadcast_to(d00, (N, C)), jnp.broadcast_to(d01, (N, C))], 1)
    den1 = jnp.concatenate([jnp.broadcast_to(d10, (N, C)), jnp.broadcast_to(d11, (N, C))], 1)
    return jnp.concatenate([num0 / den0, num1 / den1], 1)


def _bn(x, g, b):
    mu = jnp.mean(x, 0, keepdims=True)
    var = jnp.mean((x - mu) * (x - mu), 0, keepdims=True)
    return g * (x - mu) / jnp.sqrt(var + 1e-5) + b


def _epi_body(sc_ref, b_ref, g_ref, be_ref, h_ref):
    h = _finish_layer(sc_ref) + b_ref[...]
    hn = _bn(h, g_ref[...], be_ref[...])
    h_ref[...] = jnp.where(hn > 0, hn, 0.01 * hn)


def _epi(sc, b, g, be):
    return pl.pallas_call(
        _epi_body,
        out_shape=jax.ShapeDtypeStruct((N, HC), jnp.float32),
    )(sc, b.reshape(1, HC), g.reshape(1, HC), be.reshape(1, HC))


def _fin_body(sc_ref, b_ref, g_ref, be_ref, batch_ref, fc1w_ref, fc1b_ref,
              gfc_ref, bfc_ref, fc2w_ref, fc2b_ref, gemb_ref, bemb_ref, out_ref):
    h = _finish_layer(sc_ref) + b_ref[...]
    hn = _bn(h, g_ref[...], be_ref[...])
    h = jnp.where(hn > 0, hn, 0.01 * hn)
    gi = lax.broadcasted_iota(jnp.int32, (G, N), 0)
    onehot = (gi == batch_ref[...]).astype(jnp.float32)
    sums = jnp.dot(onehot, h, preferred_element_type=jnp.float32, precision=lax.Precision.HIGHEST)
    cnt = jnp.sum(onehot, 1, keepdims=True)
    hg = sums / jnp.maximum(cnt, 1.0)
    z = jnp.dot(hg, fc1w_ref[...], preferred_element_type=jnp.float32, precision=lax.Precision.HIGHEST) + fc1b_ref[...]
    z = _bn(z, gfc_ref[...], bfc_ref[...])
    z = jnp.where(z > 0, z, 0.01 * z)
    z = jnp.dot(z, fc2w_ref[...], preferred_element_type=jnp.float32, precision=lax.Precision.HIGHEST) + fc2b_ref[...]
    z = _bn(z, gemb_ref[...], bemb_ref[...])
    nrm = jnp.maximum(jnp.sqrt(jnp.sum(z * z, 1, keepdims=True)), 1e-12)
    out_ref[...] = z / nrm


def _pool_body(h_ref, batch_ref, hg_ref):
    h = h_ref[...]
    gi = lax.broadcasted_iota(jnp.int32, (G, N), 0)
    onehot = (gi == batch_ref[...]).astype(jnp.float32)
    sums = jnp.dot(onehot, h, preferred_element_type=jnp.float32, precision=lax.Precision.HIGHEST)
    cnt = jnp.sum(onehot, 1, keepdims=True)
    hg_ref[...] = sums / jnp.maximum(cnt, 1.0)


def _pool(h, batch):
    return pl.pallas_call(
        _pool_body,
        out_shape=jax.ShapeDtypeStruct((G, HC), jnp.float32),
    )(h, batch.reshape(1, N))


def _head_body(hg_ref, fc1w_ref, fc1b_ref, gfc_ref, bfc_ref, fc2w_ref, fc2b_ref,
               gemb_ref, bemb_ref, out_ref):
    z = jnp.dot(hg_ref[...], fc1w_ref[...], preferred_element_type=jnp.float32) + fc1b_ref[...]
    z = _bn(z, gfc_ref[...], bfc_ref[...])
    z = jnp.where(z > 0, z, 0.01 * z)
    z = jnp.dot(z, fc2w_ref[...], preferred_element_type=jnp.float32) + fc2b_ref[...]
    z = _bn(z, gemb_ref[...], bemb_ref[...])
    nrm = jnp.maximum(jnp.sqrt(jnp.sum(z * z, 1, keepdims=True)), 1e-12)
    out_ref[...] = z / nrm


def _head(hg, fc1w, fc1b, gfc, bfc, fc2w, fc2b, gemb, bemb):
    return pl.pallas_call(
        _head_body,
        out_shape=jax.ShapeDtypeStruct((G, EMB), jnp.float32),
    )(hg, fc1w, fc1b.reshape(1, HID), gfc.reshape(1, HID), bfc.reshape(1, HID),
      fc2w, fc2b.reshape(1, EMB), gemb.reshape(1, EMB), bemb.reshape(1, EMB))


def _fin(sc, b, g, be, batch, fc1w, fc1b, gfc, bfc, fc2w, fc2b, gemb, bemb):
    return pl.pallas_call(
        _fin_body,
        out_shape=jax.ShapeDtypeStruct((G, EMB), jnp.float32),
    )(sc, b.reshape(1, HC), g.reshape(1, HC), be.reshape(1, HC),
      batch.reshape(1, N), fc1w, fc1b.reshape(1, HID), gfc.reshape(1, HID),
      bfc.reshape(1, HID), fc2w, fc2b.reshape(1, EMB), gemb.reshape(1, EMB),
      bemb.reshape(1, EMB))


def kernel(x, edge_index, batch, Wl0, Wr0, att0, b0, g0, be0, Wl1, Wr1, att1, b1, g1, be1,
           Wl2, Wr2, att2, b2, g2, be2, fc1_w, fc1_b, gfc, bfc, fc2_w, fc2_b, gemb, bemb):
    src = edge_index[0]
    dst = edge_index[1]
    zrows = jnp.zeros((ZPT, ACC_W), jnp.float32)
    layers = ((Wl0, Wr0, att0, b0, g0, be0),
              (Wl1, Wr1, att1, b1, g1, be1),
              (Wl2, Wr2, att2, b2, g2, be2))
    h = x
    for li, (Wl, Wr, att, bb, g, be) in enumerate(layers):
        xl2, xr2 = _pre(h, Wl, Wr)
        sc = _edge_pass(xl2.reshape(2 * N, HALF), xr2.reshape(2 * N, HALF),
                        src, dst, att.reshape(HC), zrows)
        h = _epi(sc, bb, g, be)
    hg = _pool(h, batch)
    return _head(hg, fc1_w, fc1_b, gfc, bfc, fc2_w, fc2_b, gemb, bemb)
